# baseline (speedup 1.0000x reference)
import jax
import jax.numpy as jnp
import numpy as np
from jax.experimental import pallas as pl
from jax.experimental.pallas import tpu as pltpu

STRIDE = 16
ANCHOR_SIZE = 128.0
ASPECT_RATIOS = (0.2323283, 0.63365731, 1.28478321, 3.15089189)
IMG_H, IMG_W = 800, 1216
PRE_NMS_TOP_N = 6000
POST_NMS_TOP_N = 1000
NMS_THRESH = 0.7
BBOX_XFORM_CLIP = float(np.log(1000.0 / 16.0))
H_FEAT, W_FEAT = 50, 76
A = 4
NPAD = 6144
NROWS = NPAD // 128
OUT_ROWS = 1024


def _cell_anchors():
    out = []
    for r in ASPECT_RATIOS:
        w = np.round(np.sqrt(ANCHOR_SIZE * ANCHOR_SIZE / r))
        h = np.round(w * r)
        xc = yc = (STRIDE - 1.0) / 2.0
        out.append([xc - 0.5 * (w - 1), yc - 0.5 * (h - 1), xc + 0.5 * (w - 1), yc + 0.5 * (h - 1)])
    return jnp.asarray(out, dtype=jnp.float32)


def _grid_anchors(H, W):
    base = _cell_anchors()
    sx = jnp.arange(W, dtype=jnp.float32) * STRIDE
    sy = jnp.arange(H, dtype=jnp.float32) * STRIDE
    gy, gx = jnp.meshgrid(sy, sx, indexing='ij')
    shifts = jnp.stack([gx.ravel(), gy.ravel(), gx.ravel(), gy.ravel()], axis=1)
    return (shifts[:, None, :] + base[None, :, :]).reshape(-1, 4)


def _conv_same(x, w, b):
    y = jax.lax.conv_general_dilated(x, w, (1, 1), 'SAME', dimension_numbers=('NCHW', 'OIHW', 'NCHW'))
    return y + b[None, :, None, None]


def _nms_body(s_ref, dx_ref, dy_ref, dw_ref, dh_ref,
              ax1_ref, ay1_ref, ax2_ref, ay2_ref, out_ref,
              sw_ref, x1_ref, y1_ref, x2_ref, y2_ref, ar_ref, sc_ref):
    s0 = s_ref[...]
    ax1 = ax1_ref[...]
    ay1 = ay1_ref[...]
    ax2 = ax2_ref[...]
    ay2 = ay2_ref[...]
    tw = ax2 - ax1 + 1.0
    th = ay2 - ay1 + 1.0
    cx = ax1 + 0.5 * tw
    cy = ay1 + 0.5 * th
    dw = jnp.minimum(dw_ref[...], BBOX_XFORM_CLIP)
    dh = jnp.minimum(dh_ref[...], BBOX_XFORM_CLIP)
    px = dx_ref[...] * tw + cx
    py = dy_ref[...] * th + cy
    pw = jnp.exp(dw) * tw
    ph = jnp.exp(dh) * th
    x1 = jnp.clip(px - 0.5 * pw, 0.0, IMG_W - 1.0)
    y1 = jnp.clip(py - 0.5 * ph, 0.0, IMG_H - 1.0)
    x2 = jnp.clip(px + 0.5 * pw - 1.0, 0.0, IMG_W - 1.0)
    y2 = jnp.clip(py + 0.5 * ph - 1.0, 0.0, IMG_H - 1.0)
    areas = (x2 - x1 + 1.0) * (y2 - y1 + 1.0)
    sw_ref[...] = s0
    x1_ref[...] = x1
    y1_ref[...] = y1
    x2_ref[...] = x2
    y2_ref[...] = y2
    ar_ref[...] = areas
    sc_ref[...] = s0

    iota = (jax.lax.broadcasted_iota(jnp.int32, (NROWS, 128), 0) * 128
            + jax.lax.broadcasted_iota(jnp.int32, (NROWS, 128), 1))
    lane = jax.lax.broadcasted_iota(jnp.int32, (1, 128), 1)
    neg_inf = jnp.float32(-jnp.inf)

    def extract(ref, r, lsel):
        return jnp.max(jnp.where(lsel, ref[pl.ds(r, 1), :], neg_inf))

    lane128 = jnp.int32(128)

    def first_alive_lane(c):
        rowsw = sw_ref[pl.ds(c, 1), :]
        return jnp.min(jnp.where(rowsw > neg_inf, lane, lane128))

    def chunk_body(c, t):
        def cond(state):
            t_, l_ = state
            return jnp.logical_and(t_ < POST_NMS_TOP_N, l_ < lane128)

        def body(state):
            t_, l_ = state
            lsel = lane == l_
            bx1 = extract(x1_ref, c, lsel)
            by1 = extract(y1_ref, c, lsel)
            bx2 = extract(x2_ref, c, lsel)
            by2 = extract(y2_ref, c, lsel)
            barea = extract(ar_ref, c, lsel)
            bscore = extract(sc_ref, c, lsel)
            xx1 = jnp.maximum(bx1, x1)
            yy1 = jnp.maximum(by1, y1)
            xx2 = jnp.minimum(bx2, x2)
            yy2 = jnp.minimum(by2, y2)
            inter = jnp.maximum(0.0, xx2 - xx1 + 1.0) * jnp.maximum(0.0, yy2 - yy1 + 1.0)
            iou = inter / (barea + areas - inter)
            row = jnp.where(lane == 0, bx1,
                  jnp.where(lane == 1, by1,
                  jnp.where(lane == 2, bx2,
                  jnp.where(lane == 3, by2, bscore))))
            out_ref[pl.ds(t_, 1), :] = row
            sw_ref[...] = jnp.where(iou > NMS_THRESH, neg_inf, sw_ref[...])
            return t_ + 1, first_alive_lane(c)

        t_out, _ = jax.lax.while_loop(cond, body, (t, first_alive_lane(c)))
        return t_out

    t_done = jax.lax.fori_loop(0, NROWS, chunk_body, 0)

    l0 = lane == 0
    row0 = jnp.where(lane == 0, extract(x1_ref, 0, l0),
           jnp.where(lane == 1, extract(y1_ref, 0, l0),
           jnp.where(lane == 2, extract(x2_ref, 0, l0),
           jnp.where(lane == 3, extract(y2_ref, 0, l0), extract(sc_ref, 0, l0)))))

    def fill(t, carry):
        out_ref[pl.ds(t, 1), :] = row0
        return carry

    jax.lax.fori_loop(t_done, POST_NMS_TOP_N, fill, 0)


def kernel(features, W_conv, b_conv, W_obj, b_obj, W_reg, b_reg):
    t = jax.nn.relu(_conv_same(features, W_conv, b_conv))
    obj = _conv_same(t, W_obj, b_obj)
    reg = _conv_same(t, W_reg, b_reg)
    B, _, H, W = obj.shape
    obj = jnp.transpose(obj, (0, 2, 3, 1)).reshape(B, -1)
    reg = reg.reshape(B, A, 4, H, W)
    reg = jnp.transpose(reg, (0, 3, 4, 1, 2)).reshape(B, -1, 4)
    anchors = _grid_anchors(H, W)
    scores = jax.nn.sigmoid(obj[0])
    top_scores, top_idx = jax.lax.top_k(scores, PRE_NMS_TOP_N)
    codes = reg[0][top_idx]
    anc = anchors[top_idx]

    def pad(v, fill):
        return jnp.full((NPAD,), fill, jnp.float32).at[:PRE_NMS_TOP_N].set(v).reshape(NROWS, 128)

    args = (
        pad(top_scores, -jnp.inf),
        pad(codes[:, 0], 0.0), pad(codes[:, 1], 0.0),
        pad(codes[:, 2], 0.0), pad(codes[:, 3], 0.0),
        pad(anc[:, 0], 0.0), pad(anc[:, 1], 0.0),
        pad(anc[:, 2], 15.0), pad(anc[:, 3], 15.0),
    )
    out = pl.pallas_call(
        _nms_body,
        out_shape=jax.ShapeDtypeStruct((OUT_ROWS, 128), jnp.float32),
        scratch_shapes=[pltpu.VMEM((NROWS, 128), jnp.float32)] * 7,
    )(*args)
    return out[:POST_NMS_TOP_N, :5]


# speedup vs baseline: 1.8399x; 1.8399x over previous
import jax
import jax.numpy as jnp
import numpy as np
from jax.experimental import pallas as pl
from jax.experimental.pallas import tpu as pltpu

STRIDE = 16
ANCHOR_SIZE = 128.0
ASPECT_RATIOS = (0.2323283, 0.63365731, 1.28478321, 3.15089189)
IMG_H, IMG_W = 800, 1216
PRE_NMS_TOP_N = 6000
POST_NMS_TOP_N = 1000
OUT_PAD_ROWS = 1152
NMS_THRESH = 0.7
BBOX_XFORM_CLIP = float(np.log(1000.0 / 16.0))
H_FEAT, W_FEAT = 50, 76
A = 4
NPAD = 6144
NROWS = NPAD // 128
OUT_ROWS = 1024


def _cell_anchors():
    out = []
    for r in ASPECT_RATIOS:
        w = np.round(np.sqrt(ANCHOR_SIZE * ANCHOR_SIZE / r))
        h = np.round(w * r)
        xc = yc = (STRIDE - 1.0) / 2.0
        out.append([xc - 0.5 * (w - 1), yc - 0.5 * (h - 1), xc + 0.5 * (w - 1), yc + 0.5 * (h - 1)])
    return jnp.asarray(out, dtype=jnp.float32)


def _grid_anchors(H, W):
    base = _cell_anchors()
    sx = jnp.arange(W, dtype=jnp.float32) * STRIDE
    sy = jnp.arange(H, dtype=jnp.float32) * STRIDE
    gy, gx = jnp.meshgrid(sy, sx, indexing='ij')
    shifts = jnp.stack([gx.ravel(), gy.ravel(), gx.ravel(), gy.ravel()], axis=1)
    return (shifts[:, None, :] + base[None, :, :]).reshape(-1, 4)


def _conv_same(x, w, b):
    y = jax.lax.conv_general_dilated(x, w, (1, 1), 'SAME', dimension_numbers=('NCHW', 'OIHW', 'NCHW'))
    return y + b[None, :, None, None]


def _nms_body(s_ref, dx_ref, dy_ref, dw_ref, dh_ref,
              ax1_ref, ay1_ref, ax2_ref, ay2_ref, out_ref,
              sw_ref, x1_ref, y1_ref, x2_ref, y2_ref, ar_ref, sc_ref):
    s0 = s_ref[...]
    ax1 = ax1_ref[...]
    ay1 = ay1_ref[...]
    ax2 = ax2_ref[...]
    ay2 = ay2_ref[...]
    tw = ax2 - ax1 + 1.0
    th = ay2 - ay1 + 1.0
    cx = ax1 + 0.5 * tw
    cy = ay1 + 0.5 * th
    dw = jnp.minimum(dw_ref[...], BBOX_XFORM_CLIP)
    dh = jnp.minimum(dh_ref[...], BBOX_XFORM_CLIP)
    px = dx_ref[...] * tw + cx
    py = dy_ref[...] * th + cy
    pw = jnp.exp(dw) * tw
    ph = jnp.exp(dh) * th
    x1 = jnp.clip(px - 0.5 * pw, 0.0, IMG_W - 1.0)
    y1 = jnp.clip(py - 0.5 * ph, 0.0, IMG_H - 1.0)
    x2 = jnp.clip(px + 0.5 * pw - 1.0, 0.0, IMG_W - 1.0)
    y2 = jnp.clip(py + 0.5 * ph - 1.0, 0.0, IMG_H - 1.0)
    areas = (x2 - x1 + 1.0) * (y2 - y1 + 1.0)
    sw_ref[...] = s0
    x1_ref[...] = x1
    y1_ref[...] = y1
    x2_ref[...] = x2
    y2_ref[...] = y2
    ar_ref[...] = areas
    sc_ref[...] = s0

    lane = jax.lax.broadcasted_iota(jnp.int32, (1, 128), 1)
    neg_inf = jnp.float32(-jnp.inf)

    sub128 = jax.lax.broadcasted_iota(jnp.int32, (128, 128), 0)
    lane_sq = jax.lax.broadcasted_iota(jnp.int32, (128, 128), 1)
    k_lt_e = sub128 < lane_sq
    one_sq = jnp.float32(1.0)

    def row_bcast(ref, c):
        return jnp.broadcast_to(ref[pl.ds(c, 1), :], (128, 128))

    def transpose_sq(sq):
        diag = jnp.where(sub128 == lane_sq, sq, 0.0)
        ones = jnp.full((128, 128), one_sq)
        return jax.lax.dot_general(diag, ones, (((1,), (0,)), ((), ())),
                                   precision=jax.lax.Precision.HIGHEST,
                                   preferred_element_type=jnp.float32)

    def iou_tile(kx1, ky1, kx2, ky2, kar, c_e):
        ex1 = row_bcast(x1_ref, c_e)
        ey1 = row_bcast(y1_ref, c_e)
        ex2 = row_bcast(x2_ref, c_e)
        ey2 = row_bcast(y2_ref, c_e)
        ear = row_bcast(ar_ref, c_e)
        xx1 = jnp.maximum(kx1, ex1)
        yy1 = jnp.maximum(ky1, ey1)
        xx2 = jnp.minimum(kx2, ex2)
        yy2 = jnp.minimum(ky2, ey2)
        inter = jnp.maximum(0.0, xx2 - xx1 + 1.0) * jnp.maximum(0.0, yy2 - yy1 + 1.0)
        return inter / (kar + ear - inter)

    def counts_of(kept_f, o_tile):
        return jax.lax.dot_general(kept_f, o_tile, (((1,), (0,)), ((), ())),
                                   preferred_element_type=jnp.float32)

    row0 = jnp.where(lane == 0, x1_ref[0, 0],
           jnp.where(lane == 1, y1_ref[0, 0],
           jnp.where(lane == 2, x2_ref[0, 0],
           jnp.where(lane == 3, y2_ref[0, 0],
           jnp.where(lane == 4, sc_ref[0, 0], 0.0)))))
    row0_sq = jnp.broadcast_to(row0, (128, 128))
    for i in range(OUT_PAD_ROWS // 128):
        out_ref[pl.ds(i * 128, 128), :] = row0_sq

    def chunk_body(c, t):
        kx1 = transpose_sq(row_bcast(x1_ref, c))
        ky1 = transpose_sq(row_bcast(y1_ref, c))
        kx2 = transpose_sq(row_bcast(x2_ref, c))
        ky2 = transpose_sq(row_bcast(y2_ref, c))
        kar = transpose_sq(row_bcast(ar_ref, c))
        ksc = transpose_sq(row_bcast(sc_ref, c))

        alive = sw_ref[pl.ds(c, 1), :] > neg_inf
        self_o = jnp.where((iou_tile(kx1, ky1, kx2, ky2, kar, c) > NMS_THRESH) & k_lt_e,
                           1.0, 0.0)

        def fcond(s):
            return s[1]

        def fbody(s):
            kept, _ = s
            cnt = counts_of(kept, self_o)
            new = jnp.where(alive & (cnt == 0.0), 1.0, 0.0)
            return new, jnp.any(new != kept)

        kept, _ = jax.lax.while_loop(
            fcond, fbody, (jnp.where(alive, 1.0, 0.0), True))

        tot = jnp.sum(kept).astype(jnp.int32)

        @pl.when(t < POST_NMS_TOP_N)
        def _write():
            prefix = counts_of(kept, jnp.where(k_lt_e, 1.0, 0.0))
            onehot = jnp.where((jnp.broadcast_to(prefix, (128, 128)).astype(jnp.int32)
                                == sub128)
                               & (jnp.broadcast_to(kept, (128, 128)) > 0.0), 1.0, 0.0)
            data = jnp.where(lane_sq == 0, kx1,
                   jnp.where(lane_sq == 1, ky1,
                   jnp.where(lane_sq == 2, kx2,
                   jnp.where(lane_sq == 3, ky2,
                   jnp.where(lane_sq == 4, ksc, 0.0)))))
            rows = jax.lax.dot_general(onehot, data, (((1,), (0,)), ((), ())),
                                       precision=jax.lax.Precision.HIGHEST,
                                       preferred_element_type=jnp.float32)
            junk = jax.lax.broadcasted_iota(jnp.int32, (128, 128), 0) >= tot
            rows = rows + jnp.where(junk, jnp.broadcast_to(row0, (128, 128)), 0.0)
            out_ref[pl.ds(t, 128), :] = rows

        def cross(cp, _):
            @pl.when(t < POST_NMS_TOP_N)
            def _():
                o = jnp.where(iou_tile(kx1, ky1, kx2, ky2, kar, cp) > NMS_THRESH,
                              1.0, 0.0)
                cnt = counts_of(kept, o)
                sw_ref[pl.ds(cp, 1), :] = jnp.where(
                    cnt > 0.0, neg_inf, sw_ref[pl.ds(cp, 1), :])
            return 0

        jax.lax.fori_loop(c + 1, NROWS, cross, 0)
        return t + tot

    jax.lax.fori_loop(0, NROWS, chunk_body, 0)


def kernel(features, W_conv, b_conv, W_obj, b_obj, W_reg, b_reg):
    t = jax.nn.relu(_conv_same(features, W_conv, b_conv))
    obj = _conv_same(t, W_obj, b_obj)
    reg = _conv_same(t, W_reg, b_reg)
    B, _, H, W = obj.shape
    obj = jnp.transpose(obj, (0, 2, 3, 1)).reshape(B, -1)
    reg = reg.reshape(B, A, 4, H, W)
    reg = jnp.transpose(reg, (0, 3, 4, 1, 2)).reshape(B, -1, 4)
    anchors = _grid_anchors(H, W)
    scores = jax.nn.sigmoid(obj[0])
    top_scores, top_idx = jax.lax.top_k(scores, PRE_NMS_TOP_N)
    codes = reg[0][top_idx]
    anc = anchors[top_idx]

    def pad(v, fill):
        return jnp.full((NPAD,), fill, jnp.float32).at[:PRE_NMS_TOP_N].set(v).reshape(NROWS, 128)

    args = (
        pad(top_scores, -jnp.inf),
        pad(codes[:, 0], 0.0), pad(codes[:, 1], 0.0),
        pad(codes[:, 2], 0.0), pad(codes[:, 3], 0.0),
        pad(anc[:, 0], 0.0), pad(anc[:, 1], 0.0),
        pad(anc[:, 2], 15.0), pad(anc[:, 3], 15.0),
    )
    out = pl.pallas_call(
        _nms_body,
        out_shape=jax.ShapeDtypeStruct((OUT_PAD_ROWS, 128), jnp.float32),
        scratch_shapes=[pltpu.VMEM((NROWS, 128), jnp.float32)] * 7,
    )(*args)
    return out[:POST_NMS_TOP_N, :5]


# speedup vs baseline: 1.9437x; 1.0564x over previous
import jax
import jax.numpy as jnp
import numpy as np
from jax.experimental import pallas as pl
from jax.experimental.pallas import tpu as pltpu

STRIDE = 16
ANCHOR_SIZE = 128.0
ASPECT_RATIOS = (0.2323283, 0.63365731, 1.28478321, 3.15089189)
IMG_H, IMG_W = 800, 1216
PRE_NMS_TOP_N = 6000
POST_NMS_TOP_N = 1000
OUT_PAD_ROWS = 1152
NMS_THRESH = 0.7
BBOX_XFORM_CLIP = float(np.log(1000.0 / 16.0))
H_FEAT, W_FEAT = 50, 76
A = 4
NPAD = 6144
NROWS = NPAD // 128
OUT_ROWS = 1024


def _cell_anchors():
    out = []
    for r in ASPECT_RATIOS:
        w = np.round(np.sqrt(ANCHOR_SIZE * ANCHOR_SIZE / r))
        h = np.round(w * r)
        xc = yc = (STRIDE - 1.0) / 2.0
        out.append([xc - 0.5 * (w - 1), yc - 0.5 * (h - 1), xc + 0.5 * (w - 1), yc + 0.5 * (h - 1)])
    return jnp.asarray(out, dtype=jnp.float32)


def _grid_anchors(H, W):
    base = _cell_anchors()
    sx = jnp.arange(W, dtype=jnp.float32) * STRIDE
    sy = jnp.arange(H, dtype=jnp.float32) * STRIDE
    gy, gx = jnp.meshgrid(sy, sx, indexing='ij')
    shifts = jnp.stack([gx.ravel(), gy.ravel(), gx.ravel(), gy.ravel()], axis=1)
    return (shifts[:, None, :] + base[None, :, :]).reshape(-1, 4)


def _conv_same(x, w, b):
    y = jax.lax.conv_general_dilated(x, w, (1, 1), 'SAME', dimension_numbers=('NCHW', 'OIHW', 'NCHW'))
    return y + b[None, :, None, None]


def _nms_body(s_ref, dx_ref, dy_ref, dw_ref, dh_ref,
              ax1_ref, ay1_ref, ax2_ref, ay2_ref, out_ref,
              sw_ref, x1_ref, y1_ref, x2_ref, y2_ref, ar_ref, sc_ref):
    s0 = s_ref[...]
    ax1 = ax1_ref[...]
    ay1 = ay1_ref[...]
    ax2 = ax2_ref[...]
    ay2 = ay2_ref[...]
    tw = ax2 - ax1 + 1.0
    th = ay2 - ay1 + 1.0
    cx = ax1 + 0.5 * tw
    cy = ay1 + 0.5 * th
    dw = jnp.minimum(dw_ref[...], BBOX_XFORM_CLIP)
    dh = jnp.minimum(dh_ref[...], BBOX_XFORM_CLIP)
    px = dx_ref[...] * tw + cx
    py = dy_ref[...] * th + cy
    pw = jnp.exp(dw) * tw
    ph = jnp.exp(dh) * th
    x1 = jnp.clip(px - 0.5 * pw, 0.0, IMG_W - 1.0)
    y1 = jnp.clip(py - 0.5 * ph, 0.0, IMG_H - 1.0)
    x2 = jnp.clip(px + 0.5 * pw - 1.0, 0.0, IMG_W - 1.0)
    y2 = jnp.clip(py + 0.5 * ph - 1.0, 0.0, IMG_H - 1.0)
    areas = (x2 - x1 + 1.0) * (y2 - y1 + 1.0)
    sw_ref[...] = s0
    x1_ref[...] = x1
    y1_ref[...] = y1
    x2_ref[...] = x2
    y2_ref[...] = y2
    ar_ref[...] = areas
    sc_ref[...] = s0

    lane = jax.lax.broadcasted_iota(jnp.int32, (1, 128), 1)
    neg_inf = jnp.float32(-jnp.inf)

    sub128 = jax.lax.broadcasted_iota(jnp.int32, (128, 128), 0)
    lane_sq = jax.lax.broadcasted_iota(jnp.int32, (128, 128), 1)
    k_lt_e = sub128 < lane_sq
    one_sq = jnp.float32(1.0)

    def row_bcast(ref, c):
        return jnp.broadcast_to(ref[pl.ds(c, 1), :], (128, 128))

    def transpose_sq(sq):
        return jnp.transpose(sq, (1, 0))

    def iou_tile(kx1, ky1, kx2, ky2, kar, c_e):
        ex1 = row_bcast(x1_ref, c_e)
        ey1 = row_bcast(y1_ref, c_e)
        ex2 = row_bcast(x2_ref, c_e)
        ey2 = row_bcast(y2_ref, c_e)
        ear = row_bcast(ar_ref, c_e)
        xx1 = jnp.maximum(kx1, ex1)
        yy1 = jnp.maximum(ky1, ey1)
        xx2 = jnp.minimum(kx2, ex2)
        yy2 = jnp.minimum(ky2, ey2)
        inter = jnp.maximum(0.0, xx2 - xx1 + 1.0) * jnp.maximum(0.0, yy2 - yy1 + 1.0)
        return inter / (kar + ear - inter)

    def counts_of(kept_f, o_tile):
        return jax.lax.dot_general(kept_f, o_tile, (((1,), (0,)), ((), ())),
                                   preferred_element_type=jnp.float32)

    row0 = jnp.where(lane == 0, x1_ref[0, 0],
           jnp.where(lane == 1, y1_ref[0, 0],
           jnp.where(lane == 2, x2_ref[0, 0],
           jnp.where(lane == 3, y2_ref[0, 0],
           jnp.where(lane == 4, sc_ref[0, 0], 0.0)))))
    row0_sq = jnp.broadcast_to(row0, (128, 128))
    for i in range(OUT_PAD_ROWS // 128):
        out_ref[pl.ds(i * 128, 128), :] = row0_sq

    def chunk_body(c, t):
        kx1 = transpose_sq(row_bcast(x1_ref, c))
        ky1 = transpose_sq(row_bcast(y1_ref, c))
        kx2 = transpose_sq(row_bcast(x2_ref, c))
        ky2 = transpose_sq(row_bcast(y2_ref, c))
        kar = transpose_sq(row_bcast(ar_ref, c))
        ksc = transpose_sq(row_bcast(sc_ref, c))

        alive = sw_ref[pl.ds(c, 1), :] > neg_inf
        self_o = jnp.where((iou_tile(kx1, ky1, kx2, ky2, kar, c) > NMS_THRESH) & k_lt_e,
                           1.0, 0.0)

        def fcond(s):
            return s[1]

        def fbody(s):
            kept, _ = s
            cnt = counts_of(kept, self_o)
            new = jnp.where(alive & (cnt == 0.0), 1.0, 0.0)
            return new, jnp.any(new != kept)

        kept, _ = jax.lax.while_loop(
            fcond, fbody, (jnp.where(alive, 1.0, 0.0), True))

        tot = jnp.sum(kept).astype(jnp.int32)

        @pl.when(t < POST_NMS_TOP_N)
        def _write():
            prefix = counts_of(kept, jnp.where(k_lt_e, 1.0, 0.0))
            onehot = jnp.where((jnp.broadcast_to(prefix, (128, 128)).astype(jnp.int32)
                                == sub128)
                               & (jnp.broadcast_to(kept, (128, 128)) > 0.0), 1.0, 0.0)
            data = jnp.where(lane_sq == 0, kx1,
                   jnp.where(lane_sq == 1, ky1,
                   jnp.where(lane_sq == 2, kx2,
                   jnp.where(lane_sq == 3, ky2,
                   jnp.where(lane_sq == 4, ksc, 0.0)))))
            rows = jax.lax.dot_general(onehot, data, (((1,), (0,)), ((), ())),
                                       precision=jax.lax.Precision.HIGHEST,
                                       preferred_element_type=jnp.float32)
            junk = jax.lax.broadcasted_iota(jnp.int32, (128, 128), 0) >= tot
            rows = rows + jnp.where(junk, jnp.broadcast_to(row0, (128, 128)), 0.0)
            out_ref[pl.ds(t, 128), :] = rows

        def cross(cp, _):
            @pl.when(t < POST_NMS_TOP_N)
            def _():
                o = jnp.where(iou_tile(kx1, ky1, kx2, ky2, kar, cp) > NMS_THRESH,
                              1.0, 0.0)
                cnt = counts_of(kept, o)
                sw_ref[pl.ds(cp, 1), :] = jnp.where(
                    cnt > 0.0, neg_inf, sw_ref[pl.ds(cp, 1), :])
            return 0

        jax.lax.fori_loop(c + 1, NROWS, cross, 0)
        return t + tot

    jax.lax.fori_loop(0, NROWS, chunk_body, 0)


def kernel(features, W_conv, b_conv, W_obj, b_obj, W_reg, b_reg):
    t = jax.nn.relu(_conv_same(features, W_conv, b_conv))
    obj = _conv_same(t, W_obj, b_obj)
    reg = _conv_same(t, W_reg, b_reg)
    B, _, H, W = obj.shape
    obj = jnp.transpose(obj, (0, 2, 3, 1)).reshape(B, -1)
    reg = reg.reshape(B, A, 4, H, W)
    reg = jnp.transpose(reg, (0, 3, 4, 1, 2)).reshape(B, -1, 4)
    anchors = _grid_anchors(H, W)
    scores = jax.nn.sigmoid(obj[0])
    top_scores, top_idx = jax.lax.top_k(scores, PRE_NMS_TOP_N)
    codes = reg[0][top_idx]
    anc = anchors[top_idx]

    def pad(v, fill):
        return jnp.full((NPAD,), fill, jnp.float32).at[:PRE_NMS_TOP_N].set(v).reshape(NROWS, 128)

    args = (
        pad(top_scores, -jnp.inf),
        pad(codes[:, 0], 0.0), pad(codes[:, 1], 0.0),
        pad(codes[:, 2], 0.0), pad(codes[:, 3], 0.0),
        pad(anc[:, 0], 0.0), pad(anc[:, 1], 0.0),
        pad(anc[:, 2], 15.0), pad(anc[:, 3], 15.0),
    )
    out = pl.pallas_call(
        _nms_body,
        out_shape=jax.ShapeDtypeStruct((OUT_PAD_ROWS, 128), jnp.float32),
        scratch_shapes=[pltpu.VMEM((NROWS, 128), jnp.float32)] * 7,
    )(*args)
    return out[:POST_NMS_TOP_N, :5]
